# baseline (speedup 1.0000x reference)
import functools

import jax
import jax.numpy as jnp
from jax import lax
from jax.experimental import pallas as pl
from jax.experimental.pallas import tpu as pltpu
from jax.experimental.pallas import tpu_sc as plsc

D = 32
B = 16384
NC = 2
NS = 16
NW = NC * NS
BPW = B // NW

_mesh = plsc.VectorSubcoreMesh(core_axis_name="c", subcore_axis_name="s")


@functools.partial(
    pl.kernel,
    mesh=_mesh,
    out_type=jax.ShapeDtypeStruct((B, D), jnp.float32),
    scratch_types=[
        pltpu.VMEM((BPW,), jnp.int32),
        pltpu.VMEM((BPW, D), jnp.float32),
        pltpu.VMEM((BPW * 3,), jnp.float32),
        pltpu.VMEM((3, D), jnp.float32),
        pltpu.VMEM((D,), jnp.float32),
        pltpu.SemaphoreType.DMA,
    ],
    compiler_params=pltpu.CompilerParams(use_tc_tiling_on_sc=False),
)
def _sc_embed(ids_hbm, coords_hbm, table_hbm, wt_hbm, b_hbm, out_hbm,
              idx_v, rows_v, coords_v, wt_v, b_v, sem):
    wid = lax.axis_index("s") * NC + lax.axis_index("c")
    base = wid * BPW

    pltpu.sync_copy(ids_hbm.at[pl.ds(base, BPW)], idx_v)
    gather = pltpu.async_copy(table_hbm.at[idx_v], rows_v, sem)

    pltpu.sync_copy(coords_hbm.at[pl.ds(base * 3, BPW * 3)], coords_v)
    pltpu.sync_copy(wt_hbm, wt_v)
    pltpu.sync_copy(b_hbm, b_v)

    w0_lo = wt_v[0, pl.ds(0, 16)]
    w0_hi = wt_v[0, pl.ds(16, 16)]
    w1_lo = wt_v[1, pl.ds(0, 16)]
    w1_hi = wt_v[1, pl.ds(16, 16)]
    w2_lo = wt_v[2, pl.ds(0, 16)]
    w2_hi = wt_v[2, pl.ds(16, 16)]
    b_lo = b_v[pl.ds(0, 16)]
    b_hi = b_v[pl.ds(16, 16)]

    gather.wait()

    def body(g, _):
        cbase = g * 48
        cv = (coords_v[pl.ds(cbase, 16)],
              coords_v[pl.ds(cbase + 16, 16)],
              coords_v[pl.ds(cbase + 32, 16)])
        for j in range(16):
            r = g * 16 + j
            k = 3 * j
            c0 = cv[k // 16][k % 16]
            c1 = cv[(k + 1) // 16][(k + 1) % 16]
            c2 = cv[(k + 2) // 16][(k + 2) % 16]
            p_lo = c0 * w0_lo + c1 * w1_lo + c2 * w2_lo + b_lo
            p_hi = c0 * w0_hi + c1 * w1_hi + c2 * w2_hi + b_hi
            rows_v[r, pl.ds(0, 16)] = rows_v[r, pl.ds(0, 16)] + p_lo
            rows_v[r, pl.ds(16, 16)] = rows_v[r, pl.ds(16, 16)] + p_hi
        return 0

    lax.fori_loop(0, BPW // 16, body, 0)

    pltpu.sync_copy(rows_v, out_hbm.at[pl.ds(base, BPW)])


def kernel(region_ids, spatial_coords, table, W, b):
    ids = region_ids.astype(jnp.int32)
    wt = W.T
    coords_flat = spatial_coords.reshape(-1)
    return _sc_embed(ids, coords_flat, table, wt, b)
